# baseline (speedup 1.0000x reference)
import functools

import jax
import jax.numpy as jnp
from jax import lax
from jax.experimental import pallas as pl
from jax.experimental.pallas import tpu as pltpu
from jax.experimental.pallas import tpu_sc as plsc

N = 100000
HIDDEN = 128
NUM_CORES = 2
NUM_SUBCORES = 16
NW = NUM_CORES * NUM_SUBCORES
BLK = 320
STEPS = 10
NBUF = 3
GCHUNKS = (128, 128, 64)


def _sc_lookup(idx, table):
  mesh = plsc.VectorSubcoreMesh(core_axis_name="c", subcore_axis_name="s")

  @functools.partial(
      pl.kernel,
      mesh=mesh,
      out_type=jax.ShapeDtypeStruct((N, HIDDEN), jnp.float32),
      scratch_types=[
          pltpu.VMEM_SHARED((20, HIDDEN), jnp.float32),
          pltpu.VMEM((BLK,), jnp.int32),
          pltpu.VMEM((BLK,), jnp.int32),
          pltpu.VMEM((BLK,), jnp.int32),
          pltpu.VMEM((NBUF, BLK, HIDDEN), jnp.float32),
          pltpu.SemaphoreType.DMA,
          pltpu.SemaphoreType.DMA,
          pltpu.SemaphoreType.DMA,
          pltpu.SemaphoreType.DMA,
          pltpu.SemaphoreType.DMA,
          pltpu.SemaphoreType.DMA,
          pltpu.SemaphoreType.DMA,
          pltpu.SemaphoreType.DMA,
          pltpu.SemaphoreType.DMA,
      ],
  )
  def k(idx_hbm, table_hbm, out_hbm, table_v, idx_v0, idx_v1, idx_v2,
        rows_v, isem0, isem1, isem2, gsem0, gsem1, gsem2,
        ssem0, ssem1, ssem2):
    wid = lax.axis_index("s") * NUM_CORES + lax.axis_index("c")
    idx_bufs = (idx_v0, idx_v1, idx_v2)
    isems = (isem0, isem1, isem2)
    gsems = (gsem0, gsem1, gsem2)
    ssems = (ssem0, ssem1, ssem2)

    @pl.when(lax.axis_index("s") == 0)
    def _():
      pltpu.sync_copy(table_hbm, table_v)

    plsc.subcore_barrier()

    def base_of(step):
      return jnp.minimum((wid + step * NW) * BLK, N - BLK)

    idx_h = [None] * NBUF
    store_h = [None] * NBUF
    gath_h = [None] * NBUF

    idx_h[0] = pltpu.async_copy(
        idx_hbm.at[pl.ds(base_of(0), BLK)], idx_bufs[0], isems[0])

    for step in range(STEPS + 1):
      if step < STEPS:
        buf = step % NBUF
        if step + 1 < STEPS:
          idx_h[(step + 1) % NBUF] = pltpu.async_copy(
              idx_hbm.at[pl.ds(base_of(step + 1), BLK)],
              idx_bufs[(step + 1) % NBUF], isems[(step + 1) % NBUF])
        idx_h[buf].wait()
        if store_h[buf] is not None:
          store_h[buf].wait()
        ghs = []
        off = 0
        for g in GCHUNKS:
          ghs.append(pltpu.async_copy(
              table_v.at[idx_bufs[buf].at[pl.ds(off, g)]],
              rows_v.at[buf, pl.ds(off, g)],
              gsems[buf]))
          off += g
        gath_h[buf] = ghs
      if step >= 1:
        pbuf = (step - 1) % NBUF
        for h in gath_h[pbuf]:
          h.wait()
        store_h[pbuf] = pltpu.async_copy(
            rows_v.at[pbuf], out_hbm.at[pl.ds(base_of(step - 1), BLK)],
            ssems[pbuf])

    for h in store_h:
      if h is not None:
        h.wait()

  return k(idx, table)


def kernel(node_degree, degree_emb):
  return _sc_lookup(node_degree.astype(jnp.int32), degree_emb)


# speedup vs baseline: 1.0306x; 1.0306x over previous
import functools

import jax
import jax.numpy as jnp
from jax import lax
from jax.experimental import pallas as pl
from jax.experimental.pallas import tpu as pltpu
from jax.experimental.pallas import tpu_sc as plsc

N = 100000
HIDDEN = 128
NUM_CORES = 2
NUM_SUBCORES = 16
NW = NUM_CORES * NUM_SUBCORES
BLK = 392
STEPS = 8
GCHUNKS = (128, 128, 128, 8)


def _sc_lookup(idx, table):
  mesh = plsc.VectorSubcoreMesh(core_axis_name="c", subcore_axis_name="s")

  @functools.partial(
      pl.kernel,
      mesh=mesh,
      out_type=jax.ShapeDtypeStruct((N, HIDDEN), jnp.float32),
      scratch_types=[
          pltpu.VMEM_SHARED((20, HIDDEN), jnp.float32),
          pltpu.VMEM((BLK,), jnp.int32),
          pltpu.VMEM((BLK,), jnp.int32),
          pltpu.VMEM((2, BLK, HIDDEN), jnp.float32),
          pltpu.SemaphoreType.DMA,
          pltpu.SemaphoreType.DMA,
          pltpu.SemaphoreType.DMA,
          pltpu.SemaphoreType.DMA,
          pltpu.SemaphoreType.DMA,
          pltpu.SemaphoreType.DMA,
      ],
  )
  def k(idx_hbm, table_hbm, out_hbm, table_v, idx_v0, idx_v1, rows_v,
        isem0, isem1, gsem0, gsem1, ssem0, ssem1):
    wid = lax.axis_index("s") * NUM_CORES + lax.axis_index("c")
    @pl.when(lax.axis_index("s") == 0)
    def _():
      pltpu.sync_copy(table_hbm, table_v)

    plsc.subcore_barrier()
    idx_bufs = (idx_v0, idx_v1)
    isems = (isem0, isem1)
    gsems = (gsem0, gsem1)
    ssems = (ssem0, ssem1)

    def base_of(step):
      return jnp.minimum((wid + step * NW) * BLK, N - BLK)

    idx_h = [None, None]
    idx_h[0] = pltpu.async_copy(
        idx_hbm.at[pl.ds(base_of(0), BLK)], idx_bufs[0], isems[0])

    store_h = [None, None]
    gath_h = [None, None]
    for step in range(STEPS + 1):
      if step < STEPS:
        buf = step % 2
        if step + 1 < STEPS:
          idx_h[1 - buf] = pltpu.async_copy(
              idx_hbm.at[pl.ds(base_of(step + 1), BLK)],
              idx_bufs[1 - buf], isems[1 - buf])
        idx_h[buf].wait()
        if store_h[buf] is not None:
          store_h[buf].wait()
        ghs = []
        off = 0
        for g in GCHUNKS:
          ghs.append(pltpu.async_copy(
              table_v.at[idx_bufs[buf].at[pl.ds(off, g)]],
              rows_v.at[buf, pl.ds(off, g)],
              gsems[buf]))
          off += g
        gath_h[buf] = ghs
      if step >= 1:
        pbuf = (step - 1) % 2
        for h in gath_h[pbuf]:
          h.wait()
        store_h[pbuf] = pltpu.async_copy(
            rows_v.at[pbuf], out_hbm.at[pl.ds(base_of(step - 1), BLK)],
            ssems[pbuf])

    store_h[0].wait()
    store_h[1].wait()

  return k(idx, table)


def kernel(node_degree, degree_emb):
  return _sc_lookup(node_degree.astype(jnp.int32), degree_emb)


# speedup vs baseline: 1.0388x; 1.0080x over previous
import functools

import jax
import jax.numpy as jnp
from jax import lax
from jax.experimental import pallas as pl
from jax.experimental.pallas import tpu as pltpu
from jax.experimental.pallas import tpu_sc as plsc

N = 100000
HIDDEN = 128
NUM_CORES = 2
NUM_SUBCORES = 16
NW = NUM_CORES * NUM_SUBCORES
BLK = 392
STEPS = 8
GCHUNKS = (128, 128, 128, 8)


def _sc_lookup(idx, table):
  mesh = plsc.VectorSubcoreMesh(core_axis_name="c", subcore_axis_name="s")

  @functools.partial(
      pl.kernel,
      mesh=mesh,
      out_type=jax.ShapeDtypeStruct((N, HIDDEN), jnp.float32),
      scratch_types=[
          pltpu.VMEM_SHARED((NUM_SUBCORES * 20, HIDDEN), jnp.float32),
          pltpu.VMEM((BLK,), jnp.int32),
          pltpu.VMEM((BLK,), jnp.int32),
          pltpu.VMEM((2, BLK, HIDDEN), jnp.float32),
          pltpu.SemaphoreType.DMA,
          pltpu.SemaphoreType.DMA,
          pltpu.SemaphoreType.DMA,
          pltpu.SemaphoreType.DMA,
          pltpu.SemaphoreType.DMA,
          pltpu.SemaphoreType.DMA,
      ],
  )
  def k(idx_hbm, table_hbm, out_hbm, table_v, idx_v0, idx_v1, rows_v,
        isem0, isem1, gsem0, gsem1, ssem0, ssem1):
    sid = lax.axis_index("s")
    wid = sid * NUM_CORES + lax.axis_index("c")
    my_table = table_v.at[pl.ds(sid * 20, 20)]
    pltpu.sync_copy(table_hbm, my_table)
    idx_bufs = (idx_v0, idx_v1)
    isems = (isem0, isem1)
    gsems = (gsem0, gsem1)
    ssems = (ssem0, ssem1)

    def base_of(step):
      return jnp.minimum((wid + step * NW) * BLK, N - BLK)

    idx_h = [None, None]
    idx_h[0] = pltpu.async_copy(
        idx_hbm.at[pl.ds(base_of(0), BLK)], idx_bufs[0], isems[0])

    store_h = [None, None]
    for step in range(STEPS):
      buf = step % 2
      nbuf = 1 - buf
      base = base_of(step)
      if step + 1 < STEPS:
        idx_h[nbuf] = pltpu.async_copy(
            idx_hbm.at[pl.ds(base_of(step + 1), BLK)],
            idx_bufs[nbuf], isems[nbuf])
      idx_h[buf].wait()
      if store_h[buf] is not None:
        store_h[buf].wait()
      ghs = []
      off = 0
      for g in GCHUNKS:
        ghs.append(pltpu.async_copy(
            my_table.at[idx_bufs[buf].at[pl.ds(off, g)]],
            rows_v.at[buf, pl.ds(off, g)],
            gsems[buf]))
        off += g
      for h in ghs:
        h.wait()
      store_h[buf] = pltpu.async_copy(
          rows_v.at[buf], out_hbm.at[pl.ds(base, BLK)], ssems[buf])

    store_h[0].wait()
    store_h[1].wait()

  return k(idx, table)


def kernel(node_degree, degree_emb):
  return _sc_lookup(node_degree.astype(jnp.int32), degree_emb)


# speedup vs baseline: 1.0542x; 1.0148x over previous
import functools

import jax
import jax.numpy as jnp
from jax import lax
from jax.experimental import pallas as pl
from jax.experimental.pallas import tpu as pltpu
from jax.experimental.pallas import tpu_sc as plsc

N = 100000
HIDDEN = 128
NUM_CORES = 2
NUM_SUBCORES = 16
NW = NUM_CORES * NUM_SUBCORES
BLK = 392
STEPS = 8
GCHUNKS = (96, 96, 96, 104)


def _sc_lookup(idx, table):
  mesh = plsc.VectorSubcoreMesh(core_axis_name="c", subcore_axis_name="s")

  @functools.partial(
      pl.kernel,
      mesh=mesh,
      out_type=jax.ShapeDtypeStruct((N, HIDDEN), jnp.float32),
      scratch_types=[
          pltpu.VMEM_SHARED((20, HIDDEN), jnp.float32),
          pltpu.VMEM((BLK,), jnp.int32),
          pltpu.VMEM((BLK,), jnp.int32),
          pltpu.VMEM((2, BLK, HIDDEN), jnp.float32),
          pltpu.SemaphoreType.DMA,
          pltpu.SemaphoreType.DMA,
          pltpu.SemaphoreType.DMA,
          pltpu.SemaphoreType.DMA,
          pltpu.SemaphoreType.DMA,
          pltpu.SemaphoreType.DMA,
          pltpu.SemaphoreType.DMA,
          pltpu.SemaphoreType.DMA,
          pltpu.SemaphoreType.DMA,
          pltpu.SemaphoreType.DMA,
          pltpu.SemaphoreType.DMA,
          pltpu.SemaphoreType.DMA,
      ],
  )
  def k(idx_hbm, table_hbm, out_hbm, table_v, idx_v0, idx_v1, rows_v,
        isem0, isem1, g00, g01, g02, g03, g10, g11, g12, g13,
        ssem0, ssem1):
    wid = lax.axis_index("s") * NUM_CORES + lax.axis_index("c")
    @pl.when(lax.axis_index("s") == 0)
    def _():
      pltpu.sync_copy(table_hbm, table_v)

    plsc.subcore_barrier()
    idx_bufs = (idx_v0, idx_v1)
    isems = (isem0, isem1)
    gsems = ((g00, g01, g02, g03), (g10, g11, g12, g13))
    ssems = (ssem0, ssem1)

    def base_of(step):
      return jnp.minimum((wid + step * NW) * BLK, N - BLK)

    idx_h = [None, None]
    idx_h[0] = pltpu.async_copy(
        idx_hbm.at[pl.ds(base_of(0), BLK)], idx_bufs[0], isems[0])

    store_h = [[], []]
    for step in range(STEPS):
      buf = step % 2
      nbuf = 1 - buf
      base = base_of(step)
      if step + 1 < STEPS:
        idx_h[nbuf] = pltpu.async_copy(
            idx_hbm.at[pl.ds(base_of(step + 1), BLK)],
            idx_bufs[nbuf], isems[nbuf])
      idx_h[buf].wait()
      for h in store_h[buf]:
        h.wait()
      ghs = []
      off = 0
      for c, g in enumerate(GCHUNKS):
        ghs.append(pltpu.async_copy(
            table_v.at[idx_bufs[buf].at[pl.ds(off, g)]],
            rows_v.at[buf, pl.ds(off, g)],
            gsems[buf][c]))
        off += g
      store_h[buf] = []
      off = 0
      for c, g in enumerate(GCHUNKS):
        ghs[c].wait()
        store_h[buf].append(pltpu.async_copy(
            rows_v.at[buf, pl.ds(off, g)],
            out_hbm.at[pl.ds(base + off, g)],
            ssems[buf]))
        off += g

    for hs in store_h:
      for h in hs:
        h.wait()

  return k(idx, table)


def kernel(node_degree, degree_emb):
  return _sc_lookup(node_degree.astype(jnp.int32), degree_emb)


# speedup vs baseline: 1.0688x; 1.0139x over previous
import functools

import jax
import jax.numpy as jnp
from jax import lax
from jax.experimental import pallas as pl
from jax.experimental.pallas import tpu as pltpu
from jax.experimental.pallas import tpu_sc as plsc

N = 100000
HIDDEN = 128
NUM_CORES = 2
NUM_SUBCORES = 16
NW = NUM_CORES * NUM_SUBCORES
BLK = 392
STEPS = 8
GCHUNKS = (392,)


def _sc_lookup(idx, table):
  mesh = plsc.VectorSubcoreMesh(core_axis_name="c", subcore_axis_name="s")

  @functools.partial(
      pl.kernel,
      mesh=mesh,
      out_type=jax.ShapeDtypeStruct((N, HIDDEN), jnp.float32),
      scratch_types=[
          pltpu.VMEM_SHARED((20, HIDDEN), jnp.float32),
          pltpu.VMEM((BLK,), jnp.int32),
          pltpu.VMEM((BLK,), jnp.int32),
          pltpu.VMEM((2, BLK, HIDDEN), jnp.float32),
          pltpu.SemaphoreType.DMA,
          pltpu.SemaphoreType.DMA,
          pltpu.SemaphoreType.DMA,
          pltpu.SemaphoreType.DMA,
          pltpu.SemaphoreType.DMA,
          pltpu.SemaphoreType.DMA,
          pltpu.SemaphoreType.DMA,
          pltpu.SemaphoreType.DMA,
          pltpu.SemaphoreType.DMA,
          pltpu.SemaphoreType.DMA,
          pltpu.SemaphoreType.DMA,
          pltpu.SemaphoreType.DMA,
      ],
  )
  def k(idx_hbm, table_hbm, out_hbm, table_v, idx_v0, idx_v1, rows_v,
        isem0, isem1, g00, g01, g02, g03, g10, g11, g12, g13,
        ssem0, ssem1):
    wid = lax.axis_index("s") * NUM_CORES + lax.axis_index("c")
    @pl.when(lax.axis_index("s") == 0)
    def _():
      pltpu.sync_copy(table_hbm, table_v)

    plsc.subcore_barrier()
    idx_bufs = (idx_v0, idx_v1)
    isems = (isem0, isem1)
    gsems = ((g00, g01, g02, g03), (g10, g11, g12, g13))
    ssems = (ssem0, ssem1)

    def base_of(step):
      return jnp.minimum((wid + step * NW) * BLK, N - BLK)

    idx_h = [None, None]
    idx_h[0] = pltpu.async_copy(
        idx_hbm.at[pl.ds(base_of(0), BLK)], idx_bufs[0], isems[0])

    store_h = [[], []]
    for step in range(STEPS):
      buf = step % 2
      nbuf = 1 - buf
      base = base_of(step)
      if step + 1 < STEPS:
        idx_h[nbuf] = pltpu.async_copy(
            idx_hbm.at[pl.ds(base_of(step + 1), BLK)],
            idx_bufs[nbuf], isems[nbuf])
      idx_h[buf].wait()
      for h in store_h[buf]:
        h.wait()
      ghs = []
      off = 0
      for c, g in enumerate(GCHUNKS):
        ghs.append(pltpu.async_copy(
            table_v.at[idx_bufs[buf].at[pl.ds(off, g)]],
            rows_v.at[buf, pl.ds(off, g)],
            gsems[buf][c]))
        off += g
      store_h[buf] = []
      off = 0
      for c, g in enumerate(GCHUNKS):
        ghs[c].wait()
        store_h[buf].append(pltpu.async_copy(
            rows_v.at[buf, pl.ds(off, g)],
            out_hbm.at[pl.ds(base + off, g)],
            ssems[buf]))
        off += g

    for hs in store_h:
      for h in hs:
        h.wait()

  return k(idx, table)


def kernel(node_degree, degree_emb):
  return _sc_lookup(node_degree.astype(jnp.int32), degree_emb)


# speedup vs baseline: 1.0726x; 1.0036x over previous
import functools

import jax
import jax.numpy as jnp
from jax import lax
from jax.experimental import pallas as pl
from jax.experimental.pallas import tpu as pltpu
from jax.experimental.pallas import tpu_sc as plsc

N = 100000
HIDDEN = 128
NUM_CORES = 2
NUM_SUBCORES = 16
NW = NUM_CORES * NUM_SUBCORES
BLK = 448
STEPS = 7
GCHUNKS = (448,)


def _sc_lookup(idx, table):
  mesh = plsc.VectorSubcoreMesh(core_axis_name="c", subcore_axis_name="s")

  @functools.partial(
      pl.kernel,
      mesh=mesh,
      out_type=jax.ShapeDtypeStruct((N, HIDDEN), jnp.float32),
      scratch_types=[
          pltpu.VMEM_SHARED((20, HIDDEN), jnp.float32),
          pltpu.VMEM((BLK,), jnp.int32),
          pltpu.VMEM((BLK,), jnp.int32),
          pltpu.VMEM((2, BLK, HIDDEN), jnp.float32),
          pltpu.SemaphoreType.DMA,
          pltpu.SemaphoreType.DMA,
          pltpu.SemaphoreType.DMA,
          pltpu.SemaphoreType.DMA,
          pltpu.SemaphoreType.DMA,
          pltpu.SemaphoreType.DMA,
          pltpu.SemaphoreType.DMA,
          pltpu.SemaphoreType.DMA,
          pltpu.SemaphoreType.DMA,
          pltpu.SemaphoreType.DMA,
          pltpu.SemaphoreType.DMA,
          pltpu.SemaphoreType.DMA,
      ],
  )
  def k(idx_hbm, table_hbm, out_hbm, table_v, idx_v0, idx_v1, rows_v,
        isem0, isem1, g00, g01, g02, g03, g10, g11, g12, g13,
        ssem0, ssem1):
    wid = lax.axis_index("s") * NUM_CORES + lax.axis_index("c")
    @pl.when(lax.axis_index("s") == 0)
    def _():
      pltpu.sync_copy(table_hbm, table_v)

    plsc.subcore_barrier()
    idx_bufs = (idx_v0, idx_v1)
    isems = (isem0, isem1)
    gsems = ((g00, g01, g02, g03), (g10, g11, g12, g13))
    ssems = (ssem0, ssem1)

    def base_of(step):
      return jnp.minimum((wid + step * NW) * BLK, N - BLK)

    idx_h = [None, None]
    idx_h[0] = pltpu.async_copy(
        idx_hbm.at[pl.ds(base_of(0), BLK)], idx_bufs[0], isems[0])

    store_h = [[], []]
    for step in range(STEPS):
      buf = step % 2
      nbuf = 1 - buf
      base = base_of(step)
      if step + 1 < STEPS:
        idx_h[nbuf] = pltpu.async_copy(
            idx_hbm.at[pl.ds(base_of(step + 1), BLK)],
            idx_bufs[nbuf], isems[nbuf])
      idx_h[buf].wait()
      for h in store_h[buf]:
        h.wait()
      ghs = []
      off = 0
      for c, g in enumerate(GCHUNKS):
        ghs.append(pltpu.async_copy(
            table_v.at[idx_bufs[buf].at[pl.ds(off, g)]],
            rows_v.at[buf, pl.ds(off, g)],
            gsems[buf][c]))
        off += g
      store_h[buf] = []
      off = 0
      for c, g in enumerate(GCHUNKS):
        ghs[c].wait()
        store_h[buf].append(pltpu.async_copy(
            rows_v.at[buf, pl.ds(off, g)],
            out_hbm.at[pl.ds(base + off, g)],
            ssems[buf]))
        off += g

    for hs in store_h:
      for h in hs:
        h.wait()

  return k(idx, table)


def kernel(node_degree, degree_emb):
  return _sc_lookup(node_degree.astype(jnp.int32), degree_emb)


# speedup vs baseline: 1.0921x; 1.0181x over previous
import functools

import jax
import jax.numpy as jnp
from jax import lax
from jax.experimental import pallas as pl
from jax.experimental.pallas import tpu as pltpu
from jax.experimental.pallas import tpu_sc as plsc

N = 100000
HIDDEN = 128
NUM_CORES = 2
NUM_SUBCORES = 16
NW = NUM_CORES * NUM_SUBCORES
BLK = 448
STEPS = 7
SPAN = BLK * STEPS


def _sc_lookup(idx, table):
  mesh = plsc.VectorSubcoreMesh(core_axis_name="c", subcore_axis_name="s")

  @functools.partial(
      pl.kernel,
      mesh=mesh,
      out_type=jax.ShapeDtypeStruct((N, HIDDEN), jnp.float32),
      scratch_types=[
          pltpu.VMEM_SHARED((20, HIDDEN), jnp.float32),
          pltpu.VMEM((SPAN,), jnp.int32),
          pltpu.VMEM((2, BLK, HIDDEN), jnp.float32),
          pltpu.SemaphoreType.DMA,
          pltpu.SemaphoreType.DMA,
          pltpu.SemaphoreType.DMA,
          pltpu.SemaphoreType.DMA,
          pltpu.SemaphoreType.DMA,
      ],
  )
  def k(idx_hbm, table_hbm, out_hbm, table_v, idx_v, rows_v,
        isem, gsem0, gsem1, ssem0, ssem1):
    wid = lax.axis_index("s") * NUM_CORES + lax.axis_index("c")
    gsems = (gsem0, gsem1)
    ssems = (ssem0, ssem1)

    start = jnp.minimum((wid * (N // NW) // 8) * 8, N - SPAN)

    idx_h = pltpu.async_copy(idx_hbm.at[pl.ds(start, SPAN)], idx_v, isem)

    @pl.when(lax.axis_index("s") == 0)
    def _():
      pltpu.sync_copy(table_hbm, table_v)

    plsc.subcore_barrier()
    idx_h.wait()

    store_h = [None, None]
    for step in range(STEPS):
      buf = step % 2
      base = start + step * BLK
      if store_h[buf] is not None:
        store_h[buf].wait()
      pltpu.async_copy(
          table_v.at[idx_v.at[pl.ds(step * BLK, BLK)]],
          rows_v.at[buf],
          gsems[buf]).wait()
      store_h[buf] = pltpu.async_copy(
          rows_v.at[buf], out_hbm.at[pl.ds(base, BLK)], ssems[buf])

    store_h[0].wait()
    store_h[1].wait()

  return k(idx, table)


def kernel(node_degree, degree_emb):
  return _sc_lookup(node_degree.astype(jnp.int32), degree_emb)
